# baseline (speedup 1.0000x reference)
import functools

import jax
import jax.numpy as jnp
from jax import lax
from jax.experimental import pallas as pl
from jax.experimental.pallas import tpu as pltpu
from jax.experimental.pallas import tpu_sc as plsc

_N = 10000
_E = 160000
_H = 128
_EDIM = 768
_NCLS = 40
_B = 4096
_EP = 163840
_NP = 10240
_EW = 64
_ERW = _EP // _EW
_TW = _ERW // 32


def _splat(v16, i):
    idx = jnp.full((16,), i, jnp.int32)
    return lax.gather(
        v16, idx[:, None],
        dimension_numbers=lax.GatherDimensionNumbers(
            offset_dims=(), collapsed_slice_dims=(0,), start_index_map=(0,)),
        slice_sizes=(1,),
        mode=lax.GatherScatterMode.PROMISE_IN_BOUNDS)


_AEB = 5120


def _pad_table(col):
    return jnp.concatenate(
        [col, jnp.zeros((_NP - _N, 1), jnp.float32)], axis=0)


def _ae_body(ea_ref, ei_ref, x_ref, w_ref, a_ref, we1_ref, e1_ref, we2_ref,
             e2_ref, o1_ref, o2_ref, src_ref, dst_ref, h_ref, as_ref, ad_ref,
             v_ref):
    i = pl.program_id(0)

    @pl.when(i == 0)
    def _():
        h = jnp.dot(x_ref[...], w_ref[...], preferred_element_type=jnp.float32)
        h_ref[...] = jnp.concatenate(
            [h, jnp.zeros((_NP - _N, _H), jnp.float32)], axis=0)
        asd = jnp.dot(h, a_ref[...], preferred_element_type=jnp.float32)
        as_ref[...] = _pad_table(asd[:, 0:1])
        ad_ref[...] = _pad_table(asd[:, 1:2])
        v_ref[...] = (
            jnp.dot(we1_ref[...], e1_ref[...],
                    preferred_element_type=jnp.float32)
            + jnp.dot(we2_ref[...], e2_ref[...],
                      preferred_element_type=jnp.float32))

    ae2 = jnp.dot(ea_ref[...], v_ref[...], preferred_element_type=jnp.float32)
    row = i * _AEB + jax.lax.broadcasted_iota(jnp.int32, (_AEB, 1), 0)
    live = row < _E
    o1_ref[...] = jnp.where(live, ae2[:, 0:1], -1e30).reshape(_AEB // _EW, _EW)
    o2_ref[...] = jnp.where(live, ae2[:, 1:2], -1e30).reshape(_AEB // _EW, _EW)
    src = ei_ref[0, :].reshape(_AEB, 1)
    dst = ei_ref[1, :].reshape(_AEB, 1)
    src_ref[...] = jnp.where(live, src, row % _N).reshape(_AEB // _EW, _EW)
    dst_ref[...] = jnp.where(live, dst,
                             _N + row % (_NP - _N)).reshape(_AEB // _EW, _EW)


def _ae(edge_attr, ei, x, w1, a1, we1, e1p, we2, e2p):
    nin = _E // _AEB
    return pl.pallas_call(
        _ae_body,
        grid=(_EP // _AEB,),
        in_specs=[
            pl.BlockSpec((_AEB, _EDIM), lambda i: (jnp.minimum(i, nin), 0)),
            pl.BlockSpec((2, _AEB), lambda i: (0, jnp.minimum(i, nin))),
            pl.BlockSpec((_N, _H), lambda i: (0, 0)),
            pl.BlockSpec((_H, _H), lambda i: (0, 0)),
            pl.BlockSpec((_H, 8), lambda i: (0, 0)),
            pl.BlockSpec((_EDIM, _H), lambda i: (0, 0)),
            pl.BlockSpec((_H, 8), lambda i: (0, 0)),
            pl.BlockSpec((_EDIM, _H), lambda i: (0, 0)),
            pl.BlockSpec((_H, 8), lambda i: (0, 0)),
        ],
        out_specs=[
            pl.BlockSpec((_AEB // _EW, _EW), lambda i: (i, 0)),
            pl.BlockSpec((_AEB // _EW, _EW), lambda i: (i, 0)),
            pl.BlockSpec((_AEB // _EW, _EW), lambda i: (i, 0)),
            pl.BlockSpec((_AEB // _EW, _EW), lambda i: (i, 0)),
            pl.BlockSpec((_NP, _H), lambda i: (0, 0)),
            pl.BlockSpec((_NP, 1), lambda i: (0, 0)),
            pl.BlockSpec((_NP, 1), lambda i: (0, 0)),
            pl.BlockSpec((_EDIM, 8), lambda i: (0, 0)),
        ],
        out_shape=[
            jax.ShapeDtypeStruct((_ERW, _EW), jnp.float32),
            jax.ShapeDtypeStruct((_ERW, _EW), jnp.float32),
            jax.ShapeDtypeStruct((_ERW, _EW), jnp.int32),
            jax.ShapeDtypeStruct((_ERW, _EW), jnp.int32),
            jax.ShapeDtypeStruct((_NP, _H), jnp.float32),
            jax.ShapeDtypeStruct((_NP, 1), jnp.float32),
            jax.ShapeDtypeStruct((_NP, 1), jnp.float32),
            jax.ShapeDtypeStruct((_EDIM, 8), jnp.float32),
        ],
    )(edge_attr, ei, x, w1, a1, we1, e1p, we2, e2p)


def _prep2_body(p0_ref, p1_ref, d0_ref, d1_ref, b_ref, w_ref, a_ref,
                h_ref, as_ref, ad_ref):
    den = d0_ref[0] + d1_ref[0]
    rec = 1.0 / (den + 1e-16)
    agg = (p0_ref[0] + p1_ref[0]) * rec
    hr = jnp.maximum(agg + b_ref[...], 0.0)
    h2 = jnp.dot(hr, w_ref[...], preferred_element_type=jnp.float32)
    h_ref[...] = h2
    asd = jnp.dot(h2, a_ref[...], preferred_element_type=jnp.float32)
    as_ref[...] = asd[:, 0:1]
    ad_ref[...] = asd[:, 1:2]


def _prep2(part, denp3, b1, w2, a2):
    blk = 1024
    return pl.pallas_call(
        _prep2_body,
        grid=(_NP // blk,),
        in_specs=[
            pl.BlockSpec((1, blk, _H), lambda i: (0, i, 0)),
            pl.BlockSpec((1, blk, _H), lambda i: (1, i, 0)),
            pl.BlockSpec((1, blk, 1), lambda i: (0, i, 0)),
            pl.BlockSpec((1, blk, 1), lambda i: (1, i, 0)),
            pl.BlockSpec((1, _H), lambda i: (0, 0)),
            pl.BlockSpec((_H, _H), lambda i: (0, 0)),
            pl.BlockSpec((_H, 8), lambda i: (0, 0)),
        ],
        out_specs=[
            pl.BlockSpec((blk, _H), lambda i: (i, 0)),
            pl.BlockSpec((blk, 1), lambda i: (i, 0)),
            pl.BlockSpec((blk, 1), lambda i: (i, 0)),
        ],
        out_shape=[
            jax.ShapeDtypeStruct((_NP, _H), jnp.float32),
            jax.ShapeDtypeStruct((_NP, 1), jnp.float32),
            jax.ShapeDtypeStruct((_NP, 1), jnp.float32),
        ],
    )(part, part, denp3, denp3, b1, w2, a2)


def _cls_body(c_ref, wp_ref, bp_ref, o_ref):
    o_ref[...] = jnp.dot(c_ref[...], wp_ref[...],
                         preferred_element_type=jnp.float32) + bp_ref[...]


def _cls(c, wp, bp):
    return pl.pallas_call(
        _cls_body,
        out_shape=jax.ShapeDtypeStruct((_B, _NCLS), jnp.float32),
    )(c, wp, bp)


_MESH = plsc.VectorSubcoreMesh(core_axis_name="c", subcore_axis_name="s")


@functools.partial(
    pl.kernel,
    mesh=_MESH,
    out_type=(
        jax.ShapeDtypeStruct((_ERW, _EW), jnp.float32),
        jax.ShapeDtypeStruct((2, _NP), jnp.float32),
    ),
    scratch_types=[
        pltpu.VMEM((_TW, _EW), jnp.int32),
        pltpu.VMEM((_TW, _EW), jnp.int32),
        pltpu.VMEM((_TW, _EW), jnp.float32),
        pltpu.VMEM((_TW, _EW), jnp.float32),
        pltpu.VMEM((_NP,), jnp.float32),
        pltpu.VMEM((_NP,), jnp.float32),
        pltpu.VMEM((640,), jnp.float32),
        pltpu.SemaphoreType.DMA,
        pltpu.VMEM_SHARED((_NP,), jnp.float32),
    ],
    compiler_params=pltpu.CompilerParams(needs_layout_passes=False),
)
def _gat_a_sc(src_hbm, dst_hbm, ae_hbm, as_hbm, ad_hbm, ex_hbm, denp_hbm,
              src_v, dst_v, ae_v, ex_v, as_v, ad_v, zd_v, sem, denom_sh):
    c = lax.axis_index("c")
    t = lax.axis_index("s")
    w = t * 2 + c

    def zd(i, _):
        zd_v[pl.ds(i * 16, 16)] = jnp.zeros((16,), jnp.float32)
        return 0
    lax.fori_loop(0, 40, zd, 0)
    pltpu.sync_copy(zd_v, denom_sh.at[pl.ds(t * 640, 640)])

    pltpu.sync_copy(src_hbm.at[pl.ds(w * _TW, _TW)], src_v)
    pltpu.sync_copy(dst_hbm.at[pl.ds(w * _TW, _TW)], dst_v)
    pltpu.sync_copy(ae_hbm.at[pl.ds(w * _TW, _TW)], ae_v)
    pltpu.sync_copy(as_hbm, as_v)
    pltpu.sync_copy(ad_hbm, ad_v)

    def row_a(r, _):
        for jj in range(4):
            sl = pl.ds(jj * 16, 16)
            g = (plsc.load_gather(as_v, [src_v[r, sl]])
                 + plsc.load_gather(ad_v, [dst_v[r, sl]])
                 + ae_v[r, sl])
            g = jnp.where(g > 0, g, 0.2 * g)
            ex_v[r, sl] = jnp.exp(g)
        return 0
    lax.fori_loop(0, _TW, row_a, 0)

    pltpu.sync_copy(ex_v, ex_hbm.at[pl.ds(w * _TW, _TW)])

    plsc.subcore_barrier()

    descs = [
        pltpu.async_copy(ex_v.at[j], denom_sh.at[dst_v.at[j]], sem, add=True)
        for j in range(_TW)
    ]
    for d in descs:
        d.wait()

    plsc.subcore_barrier()

    pltpu.sync_copy(denom_sh.at[pl.ds(t * 640, 640)],
                    denp_hbm.at[c, pl.ds(t * 640, 640)])


@functools.partial(
    pl.kernel,
    mesh=_MESH,
    out_type=jax.ShapeDtypeStruct((2, _NP, _H), jnp.float32),
    scratch_types=[
        pltpu.VMEM((_TW // 2, _EW), jnp.int32),
        pltpu.VMEM((_TW // 2, _EW), jnp.int32),
        pltpu.VMEM((_TW // 2, _EW), jnp.float32),
        pltpu.VMEM((_EW, _H), jnp.float32),
        pltpu.VMEM((_EW, _H), jnp.float32),
        pltpu.VMEM((_EW, _H), jnp.float32),
        pltpu.SemaphoreType.DMA,
        pltpu.SemaphoreType.DMA,
        pltpu.VMEM_SHARED((_NP, _H), jnp.float32),
    ],
    compiler_params=pltpu.CompilerParams(needs_layout_passes=False),
)
def _gat_b_sc(h_hbm, src_hbm, dst_hbm, ex_hbm, out_hbm,
              src_v, dst_v, ex_v, rows0, rows1, rows2, sem_g, sem_s,
              accum_sh):
    c = lax.axis_index("c")
    t = lax.axis_index("s")
    w = t * 2 + c
    bufs = (rows0, rows1, rows2)
    nch = _TW // 2

    def zrow(i, _):
        for s2 in range(8):
            rows0[i, pl.ds(s2 * 16, 16)] = jnp.zeros((16,), jnp.float32)
        return 0
    lax.fori_loop(0, _EW, zrow, 0)
    for q in range(10):
        pltpu.sync_copy(rows0, accum_sh.at[pl.ds(t * 640 + q * _EW, _EW)])

    plsc.subcore_barrier()

    def chunk(ch, b):
        buf = bufs[b]
        nxt_buf = bufs[(b + 2) % 3]
        nxt = ch + 2

        @pl.when(jnp.logical_and(ch >= 1, nxt < nch))
        def _():
            pltpu.make_async_copy(
                nxt_buf, accum_sh.at[dst_v.at[ch - 1]], sem_s).wait()

        @pl.when(nxt < nch)
        def _():
            pltpu.async_copy(h_hbm.at[src_v.at[nxt]], nxt_buf, sem_g)

        pltpu.make_async_copy(h_hbm.at[src_v.at[ch]], buf, sem_g).wait()

        for jj in range(4):
            ex16 = ex_v[ch, pl.ds(jj * 16, 16)]
            for i in range(16):
                spl = _splat(ex16, i)
                e = jj * 16 + i
                for s2 in range(8):
                    sl2 = pl.ds(s2 * 16, 16)
                    buf[e, sl2] = buf[e, sl2] * spl

        pltpu.async_copy(buf, accum_sh.at[dst_v.at[ch]], sem_s, add=True)

    def half(hh, _):
        base = w * _TW + hh * nch
        pltpu.sync_copy(src_hbm.at[pl.ds(base, nch)], src_v)
        pltpu.sync_copy(dst_hbm.at[pl.ds(base, nch)], dst_v)
        pltpu.sync_copy(ex_hbm.at[pl.ds(base, nch)], ex_v)

        pltpu.async_copy(h_hbm.at[src_v.at[0]], rows0, sem_g)
        pltpu.async_copy(h_hbm.at[src_v.at[1]], rows1, sem_g)

        def iter3(i, _):
            for b in range(3):
                ch = i * 3 + b

                @pl.when(ch < nch)
                def _():
                    chunk(ch, b)
            return 0
        lax.fori_loop(0, (nch + 2) // 3, iter3, 0)

        for rr in (nch - 3, nch - 2, nch - 1):
            pltpu.make_async_copy(
                bufs[rr % 3], accum_sh.at[dst_v.at[rr]], sem_s).wait()
        return 0
    lax.fori_loop(0, 2, half, 0)

    plsc.subcore_barrier()

    pltpu.sync_copy(accum_sh.at[pl.ds(t * 640, 640)],
                    out_hbm.at[c, pl.ds(t * 640, 640)])


@functools.partial(
    pl.kernel,
    mesh=_MESH,
    out_type=jax.ShapeDtypeStruct((_B, _H), jnp.float32),
    scratch_types=[
        pltpu.VMEM((128,), jnp.int32),
        pltpu.VMEM((128,), jnp.int32),
        pltpu.VMEM((128, _H), jnp.float32),
        pltpu.VMEM((128, _H), jnp.float32),
        pltpu.VMEM((128, _H), jnp.float32),
        pltpu.VMEM((128, _H), jnp.float32),
        pltpu.VMEM((_H,), jnp.float32),
        pltpu.VMEM((_NP,), jnp.float32),
        pltpu.VMEM((_NP,), jnp.float32),
        pltpu.SemaphoreType.DMA,
    ],
    compiler_params=pltpu.CompilerParams(needs_layout_passes=False),
)
def _pairs_sc(p_hbm, denp_hbm, n0_hbm, n1_hbm, b2_hbm, out_hbm,
              n0_v, n1_v, a0_v, a1_v, a2_v, a3_v, b2_v, den0_v, den1_v, sem):
    c = lax.axis_index("c")
    t = lax.axis_index("s")
    w = t * 2 + c
    pltpu.sync_copy(n0_hbm.at[w], n0_v)
    pltpu.sync_copy(n1_hbm.at[w], n1_v)
    pltpu.sync_copy(b2_hbm, b2_v)
    pltpu.sync_copy(denp_hbm.at[0], den0_v)
    pltpu.sync_copy(denp_hbm.at[1], den1_v)
    descs = [
        pltpu.async_copy(p_hbm.at[0].at[n0_v], a0_v, sem),
        pltpu.async_copy(p_hbm.at[1].at[n0_v], a1_v, sem),
        pltpu.async_copy(p_hbm.at[0].at[n1_v], a2_v, sem),
        pltpu.async_copy(p_hbm.at[1].at[n1_v], a3_v, sem),
    ]
    for d in descs:
        d.wait()

    def grp_p(g, _):
        sl16 = pl.ds(g * 16, 16)
        na = n0_v[sl16]
        nb = n1_v[sl16]
        ra = 1.0 / (plsc.load_gather(den0_v, [na])
                    + plsc.load_gather(den1_v, [na]) + 1e-16)
        rb = 1.0 / (plsc.load_gather(den0_v, [nb])
                    + plsc.load_gather(den1_v, [nb]) + 1e-16)
        for i in range(16):
            sa = _splat(ra, i)
            sb = _splat(rb, i)
            e = g * 16 + i
            for s2 in range(8):
                sl = pl.ds(s2 * 16, 16)
                t0 = (a0_v[e, sl] + a1_v[e, sl]) * sa + b2_v[sl]
                t1 = (a2_v[e, sl] + a3_v[e, sl]) * sb + b2_v[sl]
                a0_v[e, sl] = t0 * t1
        return 0
    lax.fori_loop(0, 8, grp_p, 0)

    pltpu.sync_copy(a0_v, out_hbm.at[pl.ds(w * 128, 128)])


def kernel(nodes, x, edge_index, edge_attr, W1, att_src1, att_dst1, We1,
           att_e1, b1, W2, att_src2, att_dst2, We2, att_e2, b2, Wp, bp):
    f32 = jnp.float32
    nodes = nodes.astype(jnp.int32)
    ei = edge_index.astype(jnp.int32)

    a1 = jnp.zeros((_H, 8), f32).at[:, 0].set(att_src1).at[:, 1].set(att_dst1)
    a2 = jnp.zeros((_H, 8), f32).at[:, 0].set(att_src2).at[:, 1].set(att_dst2)
    e1p = jnp.zeros((_H, 8), f32).at[:, 0].set(att_e1)
    e2p = jnp.zeros((_H, 8), f32).at[:, 1].set(att_e2)

    (aep1, aep2, srcp, dstp, h1, as1, ad1, _) = _ae(
        edge_attr, ei, x, W1, a1, We1, e1p, We2, e2p)

    def _gat(h, aep, as_t, ad_t):
        ex, denp = _gat_a_sc(srcp, dstp, aep,
                             as_t.reshape(_NP), ad_t.reshape(_NP))
        return _gat_b_sc(h, srcp, dstp, ex), denp

    part1, denp1 = _gat(h1, aep1, as1, ad1)
    h2, as2, ad2 = _prep2(part1, denp1.reshape(2, _NP, 1),
                          b1.reshape(1, _H), W2, a2)
    part2, denp2 = _gat(h2, aep2, as2, ad2)

    n0 = nodes[0].reshape(_B // 128, 128)
    n1 = nodes[1].reshape(_B // 128, 128)
    cpr = _pairs_sc(part2, denp2, n0, n1, b2)
    return _cls(cpr, Wp, bp.reshape(1, _NCLS))


# speedup vs baseline: 1.0367x; 1.0367x over previous
import functools

import jax
import jax.numpy as jnp
from jax import lax
from jax.experimental import pallas as pl
from jax.experimental.pallas import tpu as pltpu
from jax.experimental.pallas import tpu_sc as plsc

_N = 10000
_E = 160000
_H = 128
_EDIM = 768
_NCLS = 40
_B = 4096
_EP = 163840
_NP = 10240
_EW = 64
_ERW = _EP // _EW
_TW = _ERW // 32


def _splat(v16, i):
    idx = jnp.full((16,), i, jnp.int32)
    return lax.gather(
        v16, idx[:, None],
        dimension_numbers=lax.GatherDimensionNumbers(
            offset_dims=(), collapsed_slice_dims=(0,), start_index_map=(0,)),
        slice_sizes=(1,),
        mode=lax.GatherScatterMode.PROMISE_IN_BOUNDS)


_AEB = 5120


def _ae_body(ea_ref, x_ref, w_ref, a_ref, we1_ref, e1_ref, we2_ref, e2_ref,
             o1_ref, o2_ref, h_ref, asd_ref, v_ref):
    i = pl.program_id(0)

    @pl.when(i == 0)
    def _():
        h = jnp.dot(x_ref[...], w_ref[...], preferred_element_type=jnp.float32)
        h_ref[...] = h
        asd_ref[...] = jnp.dot(h, a_ref[...],
                               preferred_element_type=jnp.float32)
        v_ref[...] = (
            jnp.dot(we1_ref[...], e1_ref[...],
                    preferred_element_type=jnp.float32)
            + jnp.dot(we2_ref[...], e2_ref[...],
                      preferred_element_type=jnp.float32))

    ae2 = jnp.dot(ea_ref[...], v_ref[...], preferred_element_type=jnp.float32)
    row = i * _AEB + jax.lax.broadcasted_iota(jnp.int32, (_AEB, 1), 0)
    live = row < _E
    o1_ref[...] = jnp.where(live, ae2[:, 0:1], -1e30).reshape(_AEB // _EW, _EW)
    o2_ref[...] = jnp.where(live, ae2[:, 1:2], -1e30).reshape(_AEB // _EW, _EW)


def _ae(edge_attr, x, w1, a1, we1, e1p, we2, e2p):
    nin = _E // _AEB
    return pl.pallas_call(
        _ae_body,
        grid=(_EP // _AEB,),
        in_specs=[
            pl.BlockSpec((_AEB, _EDIM), lambda i: (jnp.minimum(i, nin), 0)),
            pl.BlockSpec((_N, _H), lambda i: (0, 0)),
            pl.BlockSpec((_H, _H), lambda i: (0, 0)),
            pl.BlockSpec((_H, 8), lambda i: (0, 0)),
            pl.BlockSpec((_EDIM, _H), lambda i: (0, 0)),
            pl.BlockSpec((_H, 8), lambda i: (0, 0)),
            pl.BlockSpec((_EDIM, _H), lambda i: (0, 0)),
            pl.BlockSpec((_H, 8), lambda i: (0, 0)),
        ],
        out_specs=[
            pl.BlockSpec((_AEB // _EW, _EW), lambda i: (i, 0)),
            pl.BlockSpec((_AEB // _EW, _EW), lambda i: (i, 0)),
            pl.BlockSpec((_N, _H), lambda i: (0, 0)),
            pl.BlockSpec((_N, 8), lambda i: (0, 0)),
            pl.BlockSpec((_EDIM, 8), lambda i: (0, 0)),
        ],
        out_shape=[
            jax.ShapeDtypeStruct((_ERW, _EW), jnp.float32),
            jax.ShapeDtypeStruct((_ERW, _EW), jnp.float32),
            jax.ShapeDtypeStruct((_N, _H), jnp.float32),
            jax.ShapeDtypeStruct((_N, 8), jnp.float32),
            jax.ShapeDtypeStruct((_EDIM, 8), jnp.float32),
        ],
    )(edge_attr, x, w1, a1, we1, e1p, we2, e2p)


def _prep2_body(p0_ref, p1_ref, d0_ref, d1_ref, b_ref, w_ref, a_ref,
                h_ref, asd_ref):
    den = d0_ref[0] + d1_ref[0]
    rec = 1.0 / (den + 1e-16)
    agg = (p0_ref[0] + p1_ref[0]) * rec
    hr = jnp.maximum(agg + b_ref[...], 0.0)
    h2 = jnp.dot(hr, w_ref[...], preferred_element_type=jnp.float32)
    h_ref[...] = h2
    asd_ref[...] = jnp.dot(h2, a_ref[...], preferred_element_type=jnp.float32)


def _prep2(part, denp3, b1, w2, a2):
    blk = 1000
    return pl.pallas_call(
        _prep2_body,
        grid=(_N // blk,),
        in_specs=[
            pl.BlockSpec((1, blk, _H), lambda i: (0, i, 0)),
            pl.BlockSpec((1, blk, _H), lambda i: (1, i, 0)),
            pl.BlockSpec((1, blk, 1), lambda i: (0, i, 0)),
            pl.BlockSpec((1, blk, 1), lambda i: (1, i, 0)),
            pl.BlockSpec((1, _H), lambda i: (0, 0)),
            pl.BlockSpec((_H, _H), lambda i: (0, 0)),
            pl.BlockSpec((_H, 8), lambda i: (0, 0)),
        ],
        out_specs=[
            pl.BlockSpec((blk, _H), lambda i: (i, 0)),
            pl.BlockSpec((blk, 8), lambda i: (i, 0)),
        ],
        out_shape=[
            jax.ShapeDtypeStruct((_N, _H), jnp.float32),
            jax.ShapeDtypeStruct((_N, 8), jnp.float32),
        ],
    )(part, part, denp3, denp3, b1, w2, a2)


def _cls_body(c_ref, wp_ref, bp_ref, o_ref):
    o_ref[...] = jnp.dot(c_ref[...], wp_ref[...],
                         preferred_element_type=jnp.float32) + bp_ref[...]


def _cls(c, wp, bp):
    return pl.pallas_call(
        _cls_body,
        out_shape=jax.ShapeDtypeStruct((_B, _NCLS), jnp.float32),
    )(c, wp, bp)


_MESH = plsc.VectorSubcoreMesh(core_axis_name="c", subcore_axis_name="s")


@functools.partial(
    pl.kernel,
    mesh=_MESH,
    out_type=(
        jax.ShapeDtypeStruct((_ERW, _EW), jnp.float32),
        jax.ShapeDtypeStruct((2, _NP), jnp.float32),
    ),
    scratch_types=[
        pltpu.VMEM((_TW, _EW), jnp.int32),
        pltpu.VMEM((_TW, _EW), jnp.int32),
        pltpu.VMEM((_TW, _EW), jnp.float32),
        pltpu.VMEM((_TW, _EW), jnp.float32),
        pltpu.VMEM((_NP,), jnp.float32),
        pltpu.VMEM((_NP,), jnp.float32),
        pltpu.VMEM((640,), jnp.float32),
        pltpu.SemaphoreType.DMA,
        pltpu.VMEM_SHARED((_NP,), jnp.float32),
    ],
    compiler_params=pltpu.CompilerParams(needs_layout_passes=False),
)
def _gat_a_sc(src_hbm, dst_hbm, ae_hbm, as_hbm, ad_hbm, ex_hbm, denp_hbm,
              src_v, dst_v, ae_v, ex_v, as_v, ad_v, zd_v, sem, denom_sh):
    c = lax.axis_index("c")
    t = lax.axis_index("s")
    w = t * 2 + c

    def zd(i, _):
        zd_v[pl.ds(i * 16, 16)] = jnp.zeros((16,), jnp.float32)
        return 0
    lax.fori_loop(0, 40, zd, 0)
    pltpu.sync_copy(zd_v, denom_sh.at[pl.ds(t * 640, 640)])

    pltpu.sync_copy(src_hbm.at[pl.ds(w * _TW, _TW)], src_v)
    pltpu.sync_copy(dst_hbm.at[pl.ds(w * _TW, _TW)], dst_v)
    pltpu.sync_copy(ae_hbm.at[pl.ds(w * _TW, _TW)], ae_v)
    pltpu.sync_copy(as_hbm, as_v)
    pltpu.sync_copy(ad_hbm, ad_v)

    def row_a(r, _):
        for jj in range(4):
            sl = pl.ds(jj * 16, 16)
            g = (plsc.load_gather(as_v, [src_v[r, sl]])
                 + plsc.load_gather(ad_v, [dst_v[r, sl]])
                 + ae_v[r, sl])
            g = jnp.where(g > 0, g, 0.2 * g)
            ex_v[r, sl] = jnp.exp(g)
        return 0
    lax.fori_loop(0, _TW, row_a, 0)

    pltpu.sync_copy(ex_v, ex_hbm.at[pl.ds(w * _TW, _TW)])

    plsc.subcore_barrier()

    descs = [
        pltpu.async_copy(ex_v.at[j], denom_sh.at[dst_v.at[j]], sem, add=True)
        for j in range(_TW)
    ]
    for d in descs:
        d.wait()

    plsc.subcore_barrier()

    pltpu.sync_copy(denom_sh.at[pl.ds(t * 640, 640)],
                    denp_hbm.at[c, pl.ds(t * 640, 640)])


@functools.partial(
    pl.kernel,
    mesh=_MESH,
    out_type=jax.ShapeDtypeStruct((2, _NP, _H), jnp.float32),
    scratch_types=[
        pltpu.VMEM((_TW // 2, _EW), jnp.int32),
        pltpu.VMEM((_TW // 2, _EW), jnp.int32),
        pltpu.VMEM((_TW // 2, _EW), jnp.float32),
        pltpu.VMEM((_EW, _H), jnp.float32),
        pltpu.VMEM((_EW, _H), jnp.float32),
        pltpu.VMEM((_EW, _H), jnp.float32),
        pltpu.SemaphoreType.DMA,
        pltpu.SemaphoreType.DMA,
        pltpu.VMEM_SHARED((_NP, _H), jnp.float32),
    ],
    compiler_params=pltpu.CompilerParams(needs_layout_passes=False),
)
def _gat_b_sc(h_hbm, src_hbm, dst_hbm, ex_hbm, out_hbm,
              src_v, dst_v, ex_v, rows0, rows1, rows2, sem_g, sem_s,
              accum_sh):
    c = lax.axis_index("c")
    t = lax.axis_index("s")
    w = t * 2 + c
    bufs = (rows0, rows1, rows2)
    nch = _TW // 2

    def zrow(i, _):
        for s2 in range(8):
            rows0[i, pl.ds(s2 * 16, 16)] = jnp.zeros((16,), jnp.float32)
        return 0
    lax.fori_loop(0, _EW, zrow, 0)
    for q in range(10):
        pltpu.sync_copy(rows0, accum_sh.at[pl.ds(t * 640 + q * _EW, _EW)])

    plsc.subcore_barrier()

    def chunk(ch, b):
        buf = bufs[b]
        nxt_buf = bufs[(b + 2) % 3]
        nxt = ch + 2

        @pl.when(jnp.logical_and(ch >= 1, nxt < nch))
        def _():
            pltpu.make_async_copy(
                nxt_buf, accum_sh.at[dst_v.at[ch - 1]], sem_s).wait()

        @pl.when(nxt < nch)
        def _():
            pltpu.async_copy(h_hbm.at[src_v.at[nxt]], nxt_buf, sem_g)

        pltpu.make_async_copy(h_hbm.at[src_v.at[ch]], buf, sem_g).wait()

        for jj in range(4):
            ex16 = ex_v[ch, pl.ds(jj * 16, 16)]
            for i in range(16):
                spl = _splat(ex16, i)
                e = jj * 16 + i
                for s2 in range(8):
                    sl2 = pl.ds(s2 * 16, 16)
                    buf[e, sl2] = buf[e, sl2] * spl

        pltpu.async_copy(buf, accum_sh.at[dst_v.at[ch]], sem_s, add=True)

    def half(hh, _):
        base = w * _TW + hh * nch
        pltpu.sync_copy(src_hbm.at[pl.ds(base, nch)], src_v)
        pltpu.sync_copy(dst_hbm.at[pl.ds(base, nch)], dst_v)
        pltpu.sync_copy(ex_hbm.at[pl.ds(base, nch)], ex_v)

        pltpu.async_copy(h_hbm.at[src_v.at[0]], rows0, sem_g)
        pltpu.async_copy(h_hbm.at[src_v.at[1]], rows1, sem_g)

        def iter3(i, _):
            for b in range(3):
                ch = i * 3 + b

                @pl.when(ch < nch)
                def _():
                    chunk(ch, b)
            return 0
        lax.fori_loop(0, (nch + 2) // 3, iter3, 0)

        for rr in (nch - 3, nch - 2, nch - 1):
            pltpu.make_async_copy(
                bufs[rr % 3], accum_sh.at[dst_v.at[rr]], sem_s).wait()
        return 0
    lax.fori_loop(0, 2, half, 0)

    plsc.subcore_barrier()

    pltpu.sync_copy(accum_sh.at[pl.ds(t * 640, 640)],
                    out_hbm.at[c, pl.ds(t * 640, 640)])


@functools.partial(
    pl.kernel,
    mesh=_MESH,
    out_type=jax.ShapeDtypeStruct((_B, _H), jnp.float32),
    scratch_types=[
        pltpu.VMEM((128,), jnp.int32),
        pltpu.VMEM((128,), jnp.int32),
        pltpu.VMEM((128, _H), jnp.float32),
        pltpu.VMEM((128, _H), jnp.float32),
        pltpu.VMEM((128, _H), jnp.float32),
        pltpu.VMEM((128, _H), jnp.float32),
        pltpu.VMEM((_H,), jnp.float32),
        pltpu.VMEM((_NP,), jnp.float32),
        pltpu.VMEM((_NP,), jnp.float32),
        pltpu.SemaphoreType.DMA,
    ],
    compiler_params=pltpu.CompilerParams(needs_layout_passes=False),
)
def _pairs_sc(p_hbm, denp_hbm, n0_hbm, n1_hbm, b2_hbm, out_hbm,
              n0_v, n1_v, a0_v, a1_v, a2_v, a3_v, b2_v, den0_v, den1_v, sem):
    c = lax.axis_index("c")
    t = lax.axis_index("s")
    w = t * 2 + c
    pltpu.sync_copy(n0_hbm.at[w], n0_v)
    pltpu.sync_copy(n1_hbm.at[w], n1_v)
    pltpu.sync_copy(b2_hbm, b2_v)
    pltpu.sync_copy(denp_hbm.at[0], den0_v)
    pltpu.sync_copy(denp_hbm.at[1], den1_v)
    descs = [
        pltpu.async_copy(p_hbm.at[0].at[n0_v], a0_v, sem),
        pltpu.async_copy(p_hbm.at[1].at[n0_v], a1_v, sem),
        pltpu.async_copy(p_hbm.at[0].at[n1_v], a2_v, sem),
        pltpu.async_copy(p_hbm.at[1].at[n1_v], a3_v, sem),
    ]
    for d in descs:
        d.wait()

    def grp_p(g, _):
        sl16 = pl.ds(g * 16, 16)
        na = n0_v[sl16]
        nb = n1_v[sl16]
        ra = 1.0 / (plsc.load_gather(den0_v, [na])
                    + plsc.load_gather(den1_v, [na]) + 1e-16)
        rb = 1.0 / (plsc.load_gather(den0_v, [nb])
                    + plsc.load_gather(den1_v, [nb]) + 1e-16)
        for i in range(16):
            sa = _splat(ra, i)
            sb = _splat(rb, i)
            e = g * 16 + i
            for s2 in range(8):
                sl = pl.ds(s2 * 16, 16)
                t0 = (a0_v[e, sl] + a1_v[e, sl]) * sa + b2_v[sl]
                t1 = (a2_v[e, sl] + a3_v[e, sl]) * sb + b2_v[sl]
                a0_v[e, sl] = t0 * t1
        return 0
    lax.fori_loop(0, 8, grp_p, 0)

    pltpu.sync_copy(a0_v, out_hbm.at[pl.ds(w * 128, 128)])


def kernel(nodes, x, edge_index, edge_attr, W1, att_src1, att_dst1, We1,
           att_e1, b1, W2, att_src2, att_dst2, We2, att_e2, b2, Wp, bp):
    f32 = jnp.float32
    nodes = nodes.astype(jnp.int32)
    ei = edge_index.astype(jnp.int32)

    a1 = jnp.zeros((_H, 8), f32).at[:, 0].set(att_src1).at[:, 1].set(att_dst1)
    a2 = jnp.zeros((_H, 8), f32).at[:, 0].set(att_src2).at[:, 1].set(att_dst2)
    e1p = jnp.zeros((_H, 8), f32).at[:, 0].set(att_e1)
    e2p = jnp.zeros((_H, 8), f32).at[:, 1].set(att_e2)

    aep1, aep2, h1, asd1, _ = _ae(edge_attr, x, W1, a1, We1, e1p, We2, e2p)

    npad = _EP - _E
    pad_src = jnp.arange(npad, dtype=jnp.int32) % _N
    pad_dst = _N + jnp.arange(npad, dtype=jnp.int32) % (_NP - _N)
    srcp = jnp.concatenate([ei[0], pad_src]).reshape(_ERW, _EW)
    dstp = jnp.concatenate([ei[1], pad_dst]).reshape(_ERW, _EW)

    def _padt(v):
        return jnp.zeros((_NP,), f32).at[:_N].set(v)

    def _gat(h, aep, asd):
        ex, denp = _gat_a_sc(srcp, dstp, aep,
                             _padt(asd[:, 0]), _padt(asd[:, 1]))
        return _gat_b_sc(h, srcp, dstp, ex), denp

    part1, denp1 = _gat(h1, aep1, asd1)
    h2, asd2 = _prep2(part1, denp1.reshape(2, _NP, 1),
                      b1.reshape(1, _H), W2, a2)
    part2, denp2 = _gat(h2, aep2, asd2)

    n0 = nodes[0].reshape(_B // 128, 128)
    n1 = nodes[1].reshape(_B // 128, 128)
    cpr = _pairs_sc(part2, denp2, n0, n1, b2)
    return _cls(cpr, Wp, bp.reshape(1, _NCLS))
